# baseline (speedup 1.0000x reference)
import functools

import jax
import jax.numpy as jnp
from jax import lax
from jax.experimental import pallas as pl
from jax.experimental.pallas import tpu as pltpu
from jax.experimental.pallas import tpu_sc as plsc

NC = 2
NS = 16
NW = NC * NS
L = 16

B = 1024
T = 200
D = 64
V = 1000000
N = B * T

WC = 512
NWIN = (V + WC - 1) // WC
TAILW = V - WC * (NWIN - 1)
WIN_BASE = NWIN // NW
WIN_REM = NWIN - NW * WIN_BASE
CAP = 8192
WCAP = 4096
IDX_CHUNKS = T // 8

_mesh = plsc.VectorSubcoreMesh(
    core_axis_name="c", subcore_axis_name="s",
    num_cores=NC, num_subcores=NS)
_params = pltpu.CompilerParams(
    use_tc_tiling_on_sc=True, needs_layout_passes=False)


@functools.partial(
    pl.kernel,
    mesh=_mesh,
    compiler_params=_params,
    out_type=jax.ShapeDtypeStruct((N, 128), jnp.float32),
    scratch_types=[
        [pltpu.VMEM((8, 8, WC), jnp.float32) for _ in range(2)],
        pltpu.VMEM((CAP + L,), jnp.int32),
        pltpu.VMEM((CAP + L,), jnp.int32),
        pltpu.VMEM((WCAP + L,), jnp.int32),
        pltpu.VMEM((WCAP + L,), jnp.int32),
        pltpu.VMEM((8, 1024), jnp.int32),
        [pltpu.VMEM((L, 128), jnp.float32) for _ in range(2)],
        [pltpu.VMEM((L,), jnp.int32) for _ in range(2)],
        [pltpu.SemaphoreType.DMA for _ in range(2)],
        [pltpu.SemaphoreType.DMA for _ in range(2)],
    ],
)
def _k1_gather(tab_hbm, idx_hbm, rec_hbm, win, ckeys, ctbs, wkeys, wtbs,
               idxc, recb, tbf, wsem, rsem):
    tid = lax.axis_index("s") * NC + lax.axis_index("c")
    w0 = tid * WIN_BASE + jnp.minimum(tid, WIN_REM)
    nwin = WIN_BASE + jnp.where(tid < WIN_REM, 1, 0)
    lo = w0 * WC
    hi = jnp.minimum((w0 + nwin) * WC, V)
    iota = lax.iota(jnp.int32, L)

    def stage_win(wi, buf):
        c0 = (w0 + wi) * WC
        full = c0 + WC <= V

        @pl.when(full)
        def _():
            for db in range(8):
                pltpu.async_copy(
                    tab_hbm.at[pl.ds(db * 8, 8), pl.ds(c0, WC)],
                    win[buf].at[db], wsem[buf])

        @pl.when(jnp.logical_not(full))
        def _():
            for db in range(8):
                pltpu.async_copy(
                    tab_hbm.at[pl.ds(db * 8, 8), pl.ds(c0, TAILW)],
                    win[buf].at[db, :, pl.ds(0, TAILW)], wsem[buf])

    def wait_win(wi, buf):
        full = (w0 + wi) * WC + WC <= V

        @pl.when(full)
        def _():
            for db in range(8):
                pltpu.make_async_copy(
                    tab_hbm.at[pl.ds(0, 8), pl.ds(0, WC)],
                    win[buf].at[db], wsem[buf]).wait()

        @pl.when(jnp.logical_not(full))
        def _():
            for db in range(8):
                pltpu.make_async_copy(
                    tab_hbm.at[pl.ds(0, 8), pl.ds(0, TAILW)],
                    win[buf].at[db, :, pl.ds(0, TAILW)], wsem[buf]).wait()

    def scan_phase():
        def chunk_body(ch, cnt):
            pltpu.sync_copy(idx_hbm.at[pl.ds(ch * 8, 8), :], idxc)

            def vreg_body(j, cnt2):
                sub = lax.shift_right_logical(j, 6)
                col = jnp.bitwise_and(j, 63) * L
                kv = idxc[sub, pl.ds(col, L)]
                tb = (ch * 8 + sub) * 1024 + col + iota
                msk = jnp.logical_and(kv >= lo, kv < hi)
                off = jnp.minimum(cnt2, CAP)
                plsc.store_compressed(ckeys.at[pl.ds(off, L)], kv, mask=msk)
                plsc.store_compressed(ctbs.at[pl.ds(off, L)], tb, mask=msk)
                return cnt2 + jnp.sum(msk.astype(jnp.int32))

            return lax.fori_loop(0, 512, vreg_body, cnt, unroll=4)

        return lax.fori_loop(0, IDX_CHUNKS, chunk_body, 0)

    def gather_group(g, wcnt, wbuf, rb, do_wait):
        @pl.when(do_wait)
        def _():
            pltpu.make_async_copy(
                recb[rb], rec_hbm.at[tbf[rb]], rsem[rb]).wait()

        last = wcnt - 1
        raw = wtbs[pl.ds(g * L, L)]
        lastv = plsc.load_gather(wtbs, [jnp.full((L,), last, jnp.int32)])
        valid = (g * L + iota) < wcnt
        tbf[rb][pl.ds(0, L)] = jnp.where(valid, raw, lastv)
        for kk in range(L):
            ki = jnp.minimum(g * L + kk, last)
            cl = plsc.load_gather(wkeys, [jnp.full((L,), ki, jnp.int32)])
            for q in range(4):
                dvec = iota + (q * L)
                dbv = lax.shift_right_logical(dvec, 3)
                sbv = jnp.bitwise_and(dvec, 7)
                val = plsc.load_gather(win[wbuf], [dbv, sbv, cl])
                recb[rb][kk, pl.ds(q * L, L)] = val
        pltpu.async_copy(recb[rb], rec_hbm.at[tbf[rb]], rsem[rb])

    def process_window(wi, wbuf, cnt):
        c0 = (w0 + wi) * WC
        c1 = jnp.minimum(c0 + WC, V)

        def filt(j, fc):
            wcnt = fc
            kv = ckeys[pl.ds(j * L, L)]
            tb = ctbs[pl.ds(j * L, L)]
            vld = (j * L + iota) < cnt
            msk = jnp.logical_and(vld,
                                  jnp.logical_and(kv >= c0, kv < c1))
            off = jnp.minimum(wcnt, WCAP)
            plsc.store_compressed(wkeys.at[pl.ds(off, L)], kv - c0,
                                  mask=msk)
            plsc.store_compressed(wtbs.at[pl.ds(off, L)], tb, mask=msk)
            return wcnt + jnp.sum(msk.astype(jnp.int32))

        nv = lax.shift_right_logical(cnt + L - 1, 4)
        wcnt = lax.fori_loop(0, nv, filt, 0)
        ng = lax.shift_right_logical(wcnt + L - 1, 4)
        npair = lax.shift_right_logical(ng + 1, 1)

        def gpair(p, _):
            g0 = p * 2
            g1 = g0 + 1
            gather_group(g0, wcnt, wbuf, 0, p >= 1)

            @pl.when(g1 < ng)
            def _():
                gather_group(g1, wcnt, wbuf, 1, p >= 1)

            return 0

        lax.fori_loop(0, npair, gpair, 0)
        for rb in range(2):
            @pl.when(ng >= rb + 1)
            def _():
                pltpu.make_async_copy(
                    recb[rb], rec_hbm.at[tbf[rb]], rsem[rb]).wait()

    cnt = scan_phase()
    stage_win(0, 0)
    npair = lax.shift_right_logical(nwin + 1, 1)

    def wpair(p, _):
        wi0 = p * 2
        wi1 = wi0 + 1

        @pl.when(wi0 + 1 < nwin)
        def _():
            stage_win(wi0 + 1, 1)

        wait_win(wi0, 0)
        process_window(wi0, 0, cnt)

        @pl.when(wi1 < nwin)
        def _():
            @pl.when(wi1 + 1 < nwin)
            def _():
                stage_win(wi1 + 1, 0)

            wait_win(wi1, 1)

        process_window(wi1, 1, jnp.where(wi1 < nwin, cnt, 0))
        return 0

    lax.fori_loop(0, npair, wpair, 0)


@functools.partial(
    pl.kernel,
    mesh=_mesh,
    compiler_params=_params,
    out_type=jax.ShapeDtypeStruct((T, D, B), jnp.float32),
    scratch_types=[
        [pltpu.VMEM((128, 128), jnp.float32) for _ in range(2)],
        [pltpu.VMEM((D, 128), jnp.float32) for _ in range(2)],
        pltpu.VMEM((8, 8, 256), jnp.float32),
        [pltpu.SemaphoreType.DMA for _ in range(2)],
        [pltpu.SemaphoreType.DMA for _ in range(2)],
    ],
)
def _k2_place(rec_hbm, pos_hbm, out_hbm, rbuf, obuf, posv, isems, osems):
    tid = lax.axis_index("s") * NC + lax.axis_index("c")
    iota = lax.iota(jnp.int32, L)
    ntask = (T * 8) // NW

    for db in range(8):
        pltpu.sync_copy(pos_hbm.at[pl.ds(db * 8, 8), :], posv.at[db])

    def fetch(k, buf):
        task = k * NW + tid
        t = lax.shift_right_logical(task, 3)
        bq = jnp.bitwise_and(task, 7)
        row0 = t * B + bq * 128
        pltpu.async_copy(
            rec_hbm.at[pl.ds(row0, 128), :], rbuf[buf], isems[buf])

    def wait_fetch(buf):
        pltpu.make_async_copy(
            rec_hbm.at[pl.ds(0, 128), :], rbuf[buf], isems[buf]).wait()

    def compute(k, buf, do_wait):
        task = k * NW + tid
        t = lax.shift_right_logical(task, 3)
        bq = jnp.bitwise_and(task, 7)
        wait_fetch(buf)

        @pl.when(do_wait)
        def _():
            pltpu.make_async_copy(
                obuf[buf], out_hbm.at[0, :, pl.ds(0, 128)],
                osems[buf]).wait()

        def d_body(d, _2):
            pv = plsc.load_gather(
                posv,
                [jnp.full((L,), lax.shift_right_logical(d, 3), jnp.int32),
                 jnp.full((L,), jnp.bitwise_and(d, 7), jnp.int32),
                 jnp.full((L,), t, jnp.int32)])
            for bb in range(8):
                val = plsc.load_gather(
                    rbuf[buf],
                    [bb * L + iota, jnp.full((L,), d, jnp.int32)])
                obuf[buf][d, pl.ds(bb * L, L)] = val + pv
            return 0

        lax.fori_loop(0, D, d_body, 0, unroll=4)
        pltpu.async_copy(
            obuf[buf], out_hbm.at[t, :, pl.ds(bq * 128, 128)], osems[buf])

    fetch(0, 0)

    def pair_body(p, _):
        k0 = p * 2
        k1 = k0 + 1
        fetch(k1, 1)
        compute(k0, 0, p >= 1)

        @pl.when(k1 + 1 < ntask)
        def _():
            fetch(k1 + 1, 0)

        compute(k1, 1, p >= 1)
        return 0

    lax.fori_loop(0, ntask // 2, pair_body, 0)
    for buf in range(2):
        pltpu.make_async_copy(
            obuf[buf], out_hbm.at[0, :, pl.ds(0, 128)], osems[buf]).wait()


def kernel(inputs, token_table, pos_table):
    tabT = token_table.T
    idxT = inputs.T.astype(jnp.int32)
    posTp = jnp.pad(pos_table.T, ((0, 0), (0, 256 - T)))
    rec = _k1_gather(tabT, idxT)
    out = _k2_place(rec, posTp)
    return out.transpose(2, 0, 1)


# speedup vs baseline: 1.2248x; 1.2248x over previous
import functools

import jax
import jax.numpy as jnp
from jax import lax
from jax.experimental import pallas as pl
from jax.experimental.pallas import tpu as pltpu
from jax.experimental.pallas import tpu_sc as plsc

NC = 2
NS = 16
NW = NC * NS
L = 16

B = 1024
T = 200
D = 64
V = 1000000
N = B * T

WC = 512
NWIN = (V + WC - 1) // WC
TAILW = V - WC * (NWIN - 1)
WIN_BASE = NWIN // NW
WIN_REM = NWIN - NW * WIN_BASE
CAP = 7168
WCAP = 4096
IDX_CHUNKS = T // 8

_mesh = plsc.VectorSubcoreMesh(
    core_axis_name="c", subcore_axis_name="s",
    num_cores=NC, num_subcores=NS)
_params = pltpu.CompilerParams(
    use_tc_tiling_on_sc=True, needs_layout_passes=False)


@functools.partial(
    pl.kernel,
    mesh=_mesh,
    compiler_params=_params,
    out_type=jax.ShapeDtypeStruct((N, 128), jnp.float32),
    scratch_types=[
        [pltpu.VMEM((8, 8, WC), jnp.float32) for _ in range(2)],
        pltpu.VMEM((CAP + L,), jnp.int32),
        pltpu.VMEM((CAP + L,), jnp.int32),
        pltpu.VMEM((CAP + L,), jnp.int32),
        pltpu.VMEM((CAP + L,), jnp.int32),
        pltpu.VMEM((WCAP + L,), jnp.int32),
        pltpu.VMEM((WCAP + L,), jnp.int32),
        pltpu.VMEM((8, 1024), jnp.int32),
        [pltpu.VMEM((L, 128), jnp.float32) for _ in range(2)],
        [pltpu.VMEM((L,), jnp.int32) for _ in range(2)],
        [pltpu.SemaphoreType.DMA for _ in range(2)],
        [pltpu.SemaphoreType.DMA for _ in range(2)],
    ],
)
def _k1_gather(tab_hbm, idx_hbm, rec_hbm, win, ckeys, ctbs, gkeys, gtbs,
               wkeys, wtbs, idxc, recb, tbf, wsem, rsem):
    tid = lax.axis_index("s") * NC + lax.axis_index("c")
    w0 = tid * WIN_BASE + jnp.minimum(tid, WIN_REM)
    nwin = WIN_BASE + jnp.where(tid < WIN_REM, 1, 0)
    lo = w0 * WC
    hi = jnp.minimum((w0 + nwin) * WC, V)
    iota = lax.iota(jnp.int32, L)

    def stage_win(wi, buf):
        c0 = (w0 + wi) * WC
        full = c0 + WC <= V

        @pl.when(full)
        def _():
            for db in range(8):
                pltpu.async_copy(
                    tab_hbm.at[pl.ds(db * 8, 8), pl.ds(c0, WC)],
                    win[buf].at[db], wsem[buf])

        @pl.when(jnp.logical_not(full))
        def _():
            for db in range(8):
                pltpu.async_copy(
                    tab_hbm.at[pl.ds(db * 8, 8), pl.ds(c0, TAILW)],
                    win[buf].at[db, :, pl.ds(0, TAILW)], wsem[buf])

    def wait_win(wi, buf):
        full = (w0 + wi) * WC + WC <= V

        @pl.when(full)
        def _():
            for db in range(8):
                pltpu.make_async_copy(
                    tab_hbm.at[pl.ds(0, 8), pl.ds(0, WC)],
                    win[buf].at[db], wsem[buf]).wait()

        @pl.when(jnp.logical_not(full))
        def _():
            for db in range(8):
                pltpu.make_async_copy(
                    tab_hbm.at[pl.ds(0, 8), pl.ds(0, TAILW)],
                    win[buf].at[db, :, pl.ds(0, TAILW)], wsem[buf]).wait()

    def scan_phase():
        def chunk_body(ch, cnt):
            pltpu.sync_copy(idx_hbm.at[pl.ds(ch * 8, 8), :], idxc)

            def vreg_body(j, cnt2):
                sub = lax.shift_right_logical(j, 6)
                col = jnp.bitwise_and(j, 63) * L
                kv = idxc[sub, pl.ds(col, L)]
                tb = (ch * 8 + sub) * 1024 + col + iota
                msk = jnp.logical_and(kv >= lo, kv < hi)
                off = jnp.minimum(cnt2, CAP)
                plsc.store_compressed(ckeys.at[pl.ds(off, L)], kv, mask=msk)
                plsc.store_compressed(ctbs.at[pl.ds(off, L)], tb, mask=msk)
                return cnt2 + jnp.sum(msk.astype(jnp.int32))

            return lax.fori_loop(0, 512, vreg_body, cnt)

        return lax.fori_loop(0, IDX_CHUNKS, chunk_body, 0)

    def gather_group(g, wcnt, wbuf, rb, do_wait):
        @pl.when(do_wait)
        def _():
            pltpu.make_async_copy(
                recb[rb], rec_hbm.at[tbf[rb]], rsem[rb]).wait()

        last = wcnt - 1
        raw = wtbs[pl.ds(g * L, L)]
        lastv = plsc.load_gather(wtbs, [jnp.full((L,), last, jnp.int32)])
        valid = (g * L + iota) < wcnt
        tbf[rb][pl.ds(0, L)] = jnp.where(valid, raw, lastv)
        for kk in range(L):
            ki = jnp.minimum(g * L + kk, last)
            cl = plsc.load_gather(wkeys, [jnp.full((L,), ki, jnp.int32)])
            for q in range(4):
                dvec = iota + (q * L)
                dbv = lax.shift_right_logical(dvec, 3)
                sbv = jnp.bitwise_and(dvec, 7)
                val = plsc.load_gather(win[wbuf], [dbv, sbv, cl])
                recb[rb][kk, pl.ds(q * L, L)] = val
        pltpu.async_copy(recb[rb], rec_hbm.at[tbf[rb]], rsem[rb])

    def process_window(wi, wbuf, cnt):
        c0 = (w0 + wi) * WC
        c1 = jnp.minimum(c0 + WC, V)

        def filt(j, fc):
            wcnt = fc
            kv = gkeys[pl.ds(j * L, L)]
            tb = gtbs[pl.ds(j * L, L)]
            vld = (j * L + iota) < cnt
            msk = jnp.logical_and(vld,
                                  jnp.logical_and(kv >= c0, kv < c1))
            off = jnp.minimum(wcnt, WCAP)
            plsc.store_compressed(wkeys.at[pl.ds(off, L)], kv - c0,
                                  mask=msk)
            plsc.store_compressed(wtbs.at[pl.ds(off, L)], tb, mask=msk)
            return wcnt + jnp.sum(msk.astype(jnp.int32))

        nv = lax.shift_right_logical(cnt + L - 1, 4)
        wcnt = lax.fori_loop(0, nv, filt, 0)
        ng = lax.shift_right_logical(wcnt + L - 1, 4)
        npair = lax.shift_right_logical(ng + 1, 1)

        def gpair(p, _):
            g0 = p * 2
            g1 = g0 + 1
            gather_group(g0, wcnt, wbuf, 0, p >= 1)

            @pl.when(g1 < ng)
            def _():
                gather_group(g1, wcnt, wbuf, 1, p >= 1)

            return 0

        lax.fori_loop(0, npair, gpair, 0)
        for rb in range(2):
            @pl.when(ng >= rb + 1)
            def _():
                pltpu.make_async_copy(
                    recb[rb], rec_hbm.at[tbf[rb]], rsem[rb]).wait()

    cnt = scan_phase()
    stage_win(0, 0)

    def build_group(glo, ghi):
        def gfilt(j, gc):
            kv = ckeys[pl.ds(j * L, L)]
            tb = ctbs[pl.ds(j * L, L)]
            vld = (j * L + iota) < cnt
            msk = jnp.logical_and(vld,
                                  jnp.logical_and(kv >= glo, kv < ghi))
            off = jnp.minimum(gc, CAP)
            plsc.store_compressed(gkeys.at[pl.ds(off, L)], kv, mask=msk)
            plsc.store_compressed(gtbs.at[pl.ds(off, L)], tb, mask=msk)
            return gc + jnp.sum(msk.astype(jnp.int32))

        nv = lax.shift_right_logical(cnt + L - 1, 4)
        return lax.fori_loop(0, nv, gfilt, 0)

    def gbody(g, _):
        gw_end = jnp.minimum((g + 1) * 8, nwin)
        glo = (w0 + g * 8) * WC
        ghi = jnp.minimum((w0 + gw_end) * WC, V)
        gcnt = build_group(glo, ghi)

        def wpair(p, _2):
            wi0 = g * 8 + p * 2
            wi1 = wi0 + 1

            @pl.when(wi0 + 1 < nwin)
            def _3():
                stage_win(wi0 + 1, 1)

            @pl.when(wi0 < nwin)
            def _3():
                wait_win(wi0, 0)

            process_window(wi0, 0, jnp.where(wi0 < nwin, gcnt, 0))

            @pl.when(wi1 < nwin)
            def _3():
                @pl.when(wi1 + 1 < nwin)
                def _4():
                    stage_win(wi1 + 1, 0)

                wait_win(wi1, 1)

            process_window(wi1, 1, jnp.where(wi1 < nwin, gcnt, 0))
            return 0

        lax.fori_loop(0, 4, wpair, 0)
        return 0

    lax.fori_loop(0, 8, gbody, 0)


@functools.partial(
    pl.kernel,
    mesh=_mesh,
    compiler_params=_params,
    out_type=jax.ShapeDtypeStruct((T, D, B), jnp.float32),
    scratch_types=[
        [pltpu.VMEM((128, 128), jnp.float32) for _ in range(2)],
        [pltpu.VMEM((D, 128), jnp.float32) for _ in range(2)],
        pltpu.VMEM((8, 8, 256), jnp.float32),
        [pltpu.SemaphoreType.DMA for _ in range(2)],
        [pltpu.SemaphoreType.DMA for _ in range(2)],
    ],
)
def _k2_place(rec_hbm, pos_hbm, out_hbm, rbuf, obuf, posv, isems, osems):
    tid = lax.axis_index("s") * NC + lax.axis_index("c")
    iota = lax.iota(jnp.int32, L)
    ntask = (T * 8) // NW

    for db in range(8):
        pltpu.sync_copy(pos_hbm.at[pl.ds(db * 8, 8), :], posv.at[db])

    def fetch(k, buf):
        task = k * NW + tid
        t = lax.shift_right_logical(task, 3)
        bq = jnp.bitwise_and(task, 7)
        row0 = t * B + bq * 128
        pltpu.async_copy(
            rec_hbm.at[pl.ds(row0, 128), :], rbuf[buf], isems[buf])

    def wait_fetch(buf):
        pltpu.make_async_copy(
            rec_hbm.at[pl.ds(0, 128), :], rbuf[buf], isems[buf]).wait()

    def compute(k, buf, do_wait):
        task = k * NW + tid
        t = lax.shift_right_logical(task, 3)
        bq = jnp.bitwise_and(task, 7)
        wait_fetch(buf)

        @pl.when(do_wait)
        def _():
            pltpu.make_async_copy(
                obuf[buf], out_hbm.at[0, :, pl.ds(0, 128)],
                osems[buf]).wait()

        def d_body(d, _2):
            pv = plsc.load_gather(
                posv,
                [jnp.full((L,), lax.shift_right_logical(d, 3), jnp.int32),
                 jnp.full((L,), jnp.bitwise_and(d, 7), jnp.int32),
                 jnp.full((L,), t, jnp.int32)])
            for bb in range(8):
                val = plsc.load_gather(
                    rbuf[buf],
                    [bb * L + iota, jnp.full((L,), d, jnp.int32)])
                obuf[buf][d, pl.ds(bb * L, L)] = val + pv
            return 0

        lax.fori_loop(0, D, d_body, 0)
        pltpu.async_copy(
            obuf[buf], out_hbm.at[t, :, pl.ds(bq * 128, 128)], osems[buf])

    fetch(0, 0)

    def pair_body(p, _):
        k0 = p * 2
        k1 = k0 + 1
        fetch(k1, 1)
        compute(k0, 0, p >= 1)

        @pl.when(k1 + 1 < ntask)
        def _():
            fetch(k1 + 1, 0)

        compute(k1, 1, p >= 1)
        return 0

    lax.fori_loop(0, ntask // 2, pair_body, 0)
    for buf in range(2):
        pltpu.make_async_copy(
            obuf[buf], out_hbm.at[0, :, pl.ds(0, 128)], osems[buf]).wait()


def kernel(inputs, token_table, pos_table):
    tabT = token_table.T
    idxT = inputs.T.astype(jnp.int32)
    posTp = jnp.pad(pos_table.T, ((0, 0), (0, 256 - T)))
    rec = _k1_gather(tabT, idxT)
    out = _k2_place(rec, posTp)
    return out.transpose(2, 0, 1)


# speedup vs baseline: 1.5609x; 1.2745x over previous
import functools

import jax
import jax.numpy as jnp
from jax import lax
from jax.experimental import pallas as pl
from jax.experimental.pallas import tpu as pltpu
from jax.experimental.pallas import tpu_sc as plsc

NUM_CORES = 2
NUM_SUBCORES = 16
NUM_WORKERS = NUM_CORES * NUM_SUBCORES
LANES = 16
NBUF = 4
ROW_UNROLL = 8


def kernel(inputs, token_table, pos_table):
    B, T = inputs.shape
    V, D = token_table.shape
    idx = inputs.astype(jnp.int32)
    seqs_per_worker = B // NUM_WORKERS
    d_regs = D // LANES

    mesh = plsc.VectorSubcoreMesh(
        core_axis_name="c", subcore_axis_name="s",
        num_cores=NUM_CORES, num_subcores=NUM_SUBCORES)

    row_bufs = [pltpu.VMEM((T, D), jnp.float32) for _ in range(NBUF)]

    @functools.partial(
        pl.kernel,
        mesh=mesh,
        compiler_params=pltpu.CompilerParams(use_tc_tiling_on_sc=False),
        out_type=jax.ShapeDtypeStruct((B, T, D), jnp.float32),
        scratch_types=[
            pltpu.VMEM((seqs_per_worker, T), jnp.int32),
            pltpu.VMEM((T, D), jnp.float32),
            row_bufs,
            [pltpu.SemaphoreType.DMA for _ in range(NBUF)],
            [pltpu.SemaphoreType.DMA for _ in range(NBUF)],
        ],
    )
    def emb_kernel(tok_hbm, pos_hbm, idx_hbm, out_hbm, idx_v, pos_v, rows,
                   gsems, ssems):
        wid = lax.axis_index("s") * NUM_CORES + lax.axis_index("c")
        first_seq = wid * seqs_per_worker
        pltpu.sync_copy(idx_hbm.at[pl.ds(first_seq, seqs_per_worker)], idx_v)
        pltpu.sync_copy(pos_hbm, pos_v)

        def start_gather(s, b):
            return pltpu.async_copy(tok_hbm.at[idx_v.at[s]], rows[b],
                                    gsems[b])

        def start_scatter(s, b):
            return pltpu.async_copy(rows[b], out_hbm.at[first_seq + s],
                                    ssems[b])

        def add_pos(b):
            def add_rows(r, _):
                r0 = r * ROW_UNROLL
                for dr in range(ROW_UNROLL):
                    for c in range(d_regs):
                        sl = pl.ds(c * LANES, LANES)
                        plsc.addupdate(rows[b].at[r0 + dr, sl],
                                       pos_v[r0 + dr, sl])
                return 0

            lax.fori_loop(0, T // ROW_UNROLL, add_rows, 0)

        gathers = [None] * seqs_per_worker
        scatters = [None] * seqs_per_worker
        for s in range(NBUF):
            gathers[s] = start_gather(s, s)
        for s in range(seqs_per_worker):
            b = s % NBUF
            gathers[s].wait()
            add_pos(b)
            scatters[s] = start_scatter(s, b)
            nxt = s + NBUF
            if nxt < seqs_per_worker:
                scatters[nxt - NBUF].wait()
                gathers[nxt] = start_gather(nxt, b)
        for s in range(seqs_per_worker - NBUF, seqs_per_worker):
            scatters[s].wait()

    out = emb_kernel(token_table, pos_table, idx)
    return out
